# baseline (speedup 1.0000x reference)
import functools

import jax
import jax.numpy as jnp
from jax import lax
from jax.experimental import pallas as pl
from jax.experimental.pallas import tpu as pltpu
from jax.experimental.pallas import tpu_sc as plsc

_NB = 16
_RADIUS = 0.05
_BLOCK = 1000

_N = 100000
_NW = 32
_A = _N // _NW
_CH = 125
_NCH = _A // _CH
_FETCH = 136
_STAGE = 3136
_NPAD = 100096


def _tc_body(f_ref, p_ref, w_ref, br_ref, out_ref):
    f = f_ref[...].astype(jnp.bfloat16)
    rel = jnp.dot(f, w_ref[...], preferred_element_type=jnp.float32)
    p = p_ref[...]
    anchor = jnp.concatenate([p] * _NB, axis=1)
    out_ref[:, : _NB * 3] = anchor + rel + br_ref[...]
    out_ref[:, _NB * 3 :] = jnp.zeros((p_ref.shape[0], 128 - _NB * 3), jnp.float32)


def _sc_body(rows_ref, batch_ref, flat_ref, outb_ref, inv_ref, outv_ref,
             stage_ref, outbv_ref):
    wid = lax.axis_index("s") * 2 + lax.axis_index("c")
    base = wid * _A

    def chunk(c, carry):
        a0 = base + c * _CH
        a0a = (a0 // 8) * 8
        da = a0 - a0a
        pltpu.sync_copy(rows_ref.at[pl.ds(a0a, _FETCH), :], inv_ref)

        def anchor(a, carry2):
            for j in range(3):
                outv_ref[pl.ds(a * 48 + j * 16, 16)] = (
                    inv_ref[a + da, pl.ds(j * 16, 16)])
            return carry2

        lax.fori_loop(0, _CH, anchor, 0)
        pltpu.sync_copy(outv_ref, flat_ref.at[pl.ds(a0 * 48, _CH * 48)])
        return carry

    lax.fori_loop(0, _NCH, chunk, 0)

    astart = (base // 8) * 8
    off = base - astart
    pltpu.sync_copy(batch_ref.at[pl.ds(astart, _STAGE)], stage_ref)

    def banchor(t, carry):
        idx = jnp.zeros((16,), jnp.int32) + (t + off)
        outbv_ref[pl.ds(t * 16, 16)] = plsc.load_gather(stage_ref, [idx])
        return carry

    lax.fori_loop(0, _A, banchor, 0)
    pltpu.sync_copy(outbv_ref, outb_ref.at[pl.ds(base * _NB, _A * _NB)])


def kernel(points, features, batch, W, b):
    n, d = features.shape
    wr = (W * _RADIUS).astype(jnp.bfloat16)
    br = (b * _RADIUS).reshape(1, _NB * 3)

    rows = pl.pallas_call(
        _tc_body,
        grid=((n + 8 + _BLOCK - 1) // _BLOCK,),
        in_specs=[
            pl.BlockSpec((_BLOCK, d), lambda i: (i, 0)),
            pl.BlockSpec((_BLOCK, 3), lambda i: (i, 0)),
            pl.BlockSpec((d, _NB * 3), lambda i: (0, 0)),
            pl.BlockSpec((1, _NB * 3), lambda i: (0, 0)),
        ],
        out_specs=pl.BlockSpec((_BLOCK, 128), lambda i: (i, 0)),
        out_shape=jax.ShapeDtypeStruct((n + 8, 128), jnp.float32),
    )(features, points, wr, br)

    batch_padded = jnp.pad(batch, (0, _NPAD - n))
    expand = functools.partial(
        pl.kernel,
        out_type=[
            jax.ShapeDtypeStruct((n * _NB * 3,), jnp.float32),
            jax.ShapeDtypeStruct((n * _NB,), batch.dtype),
        ],
        mesh=plsc.VectorSubcoreMesh(core_axis_name="c", subcore_axis_name="s"),
        compiler_params=pltpu.CompilerParams(needs_layout_passes=False),
        scratch_types=[
            pltpu.VMEM((_FETCH, 128), jnp.float32),
            pltpu.VMEM((_CH * 48,), jnp.float32),
            pltpu.VMEM((_STAGE,), jnp.int32),
            pltpu.VMEM((_A * _NB,), jnp.int32),
        ],
    )(_sc_body)
    flat, out_batch = expand(rows, batch_padded)

    return flat.reshape(n * _NB, 3), out_batch


# speedup vs baseline: 3.6756x; 3.6756x over previous
import functools

import jax
import jax.numpy as jnp
from jax import lax
from jax.experimental import pallas as pl
from jax.experimental.pallas import tpu as pltpu
from jax.experimental.pallas import tpu_sc as plsc

_NB = 16
_RADIUS = 0.05
_BLOCK = 1000

_N = 100000
_NW = 32
_A = _N // _NW
_CH = 120
_STAGE = 3136
_NPAD = 100096


def _tc_body(f_ref, p_ref, w_ref, br_ref, out_ref):
    f = f_ref[...].astype(jnp.bfloat16)
    rel = jnp.dot(f, w_ref[...], preferred_element_type=jnp.float32)
    p = p_ref[...]
    anchor = jnp.concatenate([p] * _NB, axis=1)
    out_ref[:, : _NB * 3] = anchor + rel + br_ref[...]


def _sc_body(rows_ref, batch_ref, x_ref, outb_ref, inv_ref, outv_ref,
             stage_ref, outbv_ref):
    wid = lax.axis_index("s") * 2 + lax.axis_index("c")
    s = (wid * _A) // 8 * 8
    e = ((wid + 1) * _A) // 8 * 8
    cols = [3 * lax.iota(jnp.int32, 16) + j for j in range(3)]

    def do_span(a0, nch, ch):
        def chunk(c, carry):
            ac = a0 + c * ch
            pltpu.sync_copy(rows_ref.at[pl.ds(ac, ch), :],
                            inv_ref.at[pl.ds(0, ch), :])

            def anchor(a, carry2):
                for j in range(3):
                    row = jnp.zeros((16,), jnp.int32) + a
                    v = plsc.load_gather(inv_ref, [row, cols[j]])
                    outv_ref[j, pl.ds(a * 16, 16)] = v
                return carry2

            lax.fori_loop(0, ch, anchor, 0)
            pltpu.sync_copy(outv_ref.at[:, pl.ds(0, ch * 16)],
                            x_ref.at[:, pl.ds(ac * 16, ch * 16)])
            return carry

        lax.fori_loop(0, nch, chunk, 0)

    nfull = (e - s) // _CH
    do_span(s, nfull, _CH)
    @pl.when(e - s - nfull * _CH == 8)
    def _():
        do_span(s + nfull * _CH, 1, 8)

    base = wid * _A
    astart = (base // 8) * 8
    off = base - astart
    pltpu.sync_copy(batch_ref.at[pl.ds(astart, _STAGE)], stage_ref)

    def banchor(t, carry):
        idx = jnp.zeros((16,), jnp.int32) + (t + off)
        outbv_ref[pl.ds(t * 16, 16)] = plsc.load_gather(stage_ref, [idx])
        return carry

    lax.fori_loop(0, _A, banchor, 0)
    pltpu.sync_copy(outbv_ref, outb_ref.at[pl.ds(base * _NB, _A * _NB)])


def kernel(points, features, batch, W, b):
    n, d = features.shape
    wr = (W * _RADIUS).astype(jnp.bfloat16)
    br = (b * _RADIUS).reshape(1, _NB * 3)

    rows = pl.pallas_call(
        _tc_body,
        grid=(n // _BLOCK,),
        in_specs=[
            pl.BlockSpec((_BLOCK, d), lambda i: (i, 0)),
            pl.BlockSpec((_BLOCK, 3), lambda i: (i, 0)),
            pl.BlockSpec((d, _NB * 3), lambda i: (0, 0)),
            pl.BlockSpec((1, _NB * 3), lambda i: (0, 0)),
        ],
        out_specs=pl.BlockSpec((_BLOCK, 128), lambda i: (i, 0)),
        out_shape=jax.ShapeDtypeStruct((n, 128), jnp.float32),
    )(features, points, wr, br)

    batch_padded = jnp.pad(batch, (0, _NPAD - n))
    expand = functools.partial(
        pl.kernel,
        out_type=[
            jax.ShapeDtypeStruct((3, n * _NB), jnp.float32),
            jax.ShapeDtypeStruct((n * _NB,), batch.dtype),
        ],
        mesh=plsc.VectorSubcoreMesh(core_axis_name="c", subcore_axis_name="s"),
        compiler_params=pltpu.CompilerParams(needs_layout_passes=False),
        scratch_types=[
            pltpu.VMEM((_CH, 128), jnp.float32),
            pltpu.VMEM((3, _CH * _NB), jnp.float32),
            pltpu.VMEM((_STAGE,), jnp.int32),
            pltpu.VMEM((_A * _NB,), jnp.int32),
        ],
    )(_sc_body)
    xt, out_batch = expand(rows, batch_padded)

    return xt.T, out_batch
